# Initial kernel scaffold; baseline (speedup 1.0000x reference)
import jax
import jax.numpy as jnp
from jax.experimental import pallas as pl


def kernel(source_nodes, target_nodes, node_features, W, b):
    raise NotImplementedError("write your pallas kernel here")


# speedup vs baseline: 39.0685x; 39.0685x over previous
import functools

import jax
import jax.numpy as jnp
from jax import lax
from jax.experimental import pallas as pl
from jax.experimental.pallas import tpu as pltpu
from jax.experimental.pallas import tpu_sc as plsc

N_NODES = 10000
N_EDGES = 320000
D_FEAT = 128

_NC, _NS = 2, 16
_NW = _NC * _NS
_E_PER = N_EDGES // _NW
_CHUNK = 16


def _scores_body(x_ref, w_ref, b_ref, o_ref):
    o_ref[...] = (
        jnp.dot(x_ref[...], w_ref[...], preferred_element_type=jnp.float32)
        + b_ref[...]
    )


_scores_call = pl.pallas_call(
    _scores_body,
    out_shape=jax.ShapeDtypeStruct((N_NODES, 2), jnp.float32),
)


_mesh = plsc.VectorSubcoreMesh(core_axis_name="c", subcore_axis_name="s")


@functools.partial(
    pl.kernel,
    mesh=_mesh,
    out_type=jax.ShapeDtypeStruct((N_EDGES,), jnp.float32),
    scratch_types=[
        pltpu.VMEM((2 * N_NODES,), jnp.float32),
        pltpu.VMEM((_E_PER,), jnp.int32),
        pltpu.VMEM((_E_PER,), jnp.int32),
        pltpu.VMEM((_E_PER,), jnp.float32),
    ],
    compiler_params=pltpu.CompilerParams(needs_layout_passes=False),
)
def _edge_gather(tab_hbm, src_hbm, tgt_hbm, out_hbm, tab_v, src_v, tgt_v, out_v):
    wid = lax.axis_index("s") * _NC + lax.axis_index("c")
    base = wid * _E_PER
    pltpu.sync_copy(tab_hbm, tab_v)
    pltpu.sync_copy(src_hbm.at[pl.ds(base, _E_PER)], src_v)
    pltpu.sync_copy(tgt_hbm.at[pl.ds(base, _E_PER)], tgt_v)

    def body(i, carry):
        off = pl.multiple_of(i * _CHUNK, _CHUNK)
        si = src_v[pl.ds(off, _CHUNK)]
        ti = tgt_v[pl.ds(off, _CHUNK)]
        vs = plsc.load_gather(tab_v, [si * 2])
        vt = plsc.load_gather(tab_v, [ti * 2 + 1])
        out_v[pl.ds(off, _CHUNK)] = vs + vt
        return carry

    lax.fori_loop(0, _E_PER // _CHUNK, body, 0)
    pltpu.sync_copy(out_v, out_hbm.at[pl.ds(base, _E_PER)])


def kernel(source_nodes, target_nodes, node_features, W, b):
    src = source_nodes.astype(jnp.int32)
    tgt = target_nodes.astype(jnp.int32)
    w2 = jnp.concatenate([W[:D_FEAT], W[D_FEAT:]], axis=1)
    b2 = jnp.concatenate([b, jnp.zeros((1,), jnp.float32)]).reshape(1, 2)
    scores = _scores_call(node_features, w2, b2)
    tab = scores.reshape(-1)
    return _edge_gather(tab, src, tgt)


# speedup vs baseline: 40.3794x; 1.0336x over previous
import functools

import jax
import jax.numpy as jnp
from jax import lax
from jax.experimental import pallas as pl
from jax.experimental.pallas import tpu as pltpu
from jax.experimental.pallas import tpu_sc as plsc

N_NODES = 10000
N_EDGES = 320000
D_FEAT = 128

_NC, _NS = 2, 16
_NW = _NC * _NS
_E_PER = N_EDGES // _NW
_CHUNK = 16

def _scores_body(x_ref, w0_ref, w1_ref, b_ref, o0_ref, o1_ref):
    x = x_ref[...]
    o0_ref[...] = jnp.sum(x * w0_ref[...], axis=1) + b_ref[0, 0]
    o1_ref[...] = jnp.sum(x * w1_ref[...], axis=1)


_scores_call = pl.pallas_call(
    _scores_body,
    out_shape=[
        jax.ShapeDtypeStruct((N_NODES,), jnp.float32),
        jax.ShapeDtypeStruct((N_NODES,), jnp.float32),
    ],
)


_mesh = plsc.VectorSubcoreMesh(core_axis_name="c", subcore_axis_name="s")


@functools.partial(
    pl.kernel,
    mesh=_mesh,
    out_type=jax.ShapeDtypeStruct((N_EDGES,), jnp.float32),
    scratch_types=[
        pltpu.VMEM((N_NODES,), jnp.float32),
        pltpu.VMEM((N_NODES,), jnp.float32),
        pltpu.VMEM((_E_PER,), jnp.int32),
        pltpu.VMEM((_E_PER,), jnp.int32),
        pltpu.VMEM((_E_PER,), jnp.float32),
        pltpu.SemaphoreType.DMA,
        pltpu.SemaphoreType.DMA,
        pltpu.SemaphoreType.DMA,
        pltpu.SemaphoreType.DMA,
    ],
    compiler_params=pltpu.CompilerParams(needs_layout_passes=False),
)
def _edge_gather(
    s0_hbm, s1_hbm, src_hbm, tgt_hbm, out_hbm,
    s0_v, s1_v, src_v, tgt_v, out_v, sem0, sem1, sem2, sem3,
):
    wid = lax.axis_index("s") * _NC + lax.axis_index("c")
    base = wid * _E_PER
    cp0 = pltpu.async_copy(s0_hbm, s0_v, sem0)
    cp1 = pltpu.async_copy(s1_hbm, s1_v, sem1)
    cp2 = pltpu.async_copy(src_hbm.at[pl.ds(base, _E_PER)], src_v, sem2)
    cp3 = pltpu.async_copy(tgt_hbm.at[pl.ds(base, _E_PER)], tgt_v, sem3)
    cp0.wait()
    cp1.wait()
    cp2.wait()
    cp3.wait()

    def body(i, carry):
        off = pl.multiple_of(i * _CHUNK, _CHUNK)
        si = src_v[pl.ds(off, _CHUNK)]
        ti = tgt_v[pl.ds(off, _CHUNK)]
        vs = plsc.load_gather(s0_v, [si])
        vt = plsc.load_gather(s1_v, [ti])
        out_v[pl.ds(off, _CHUNK)] = vs + vt
        return carry

    lax.fori_loop(0, _E_PER // _CHUNK, body, 0)
    pltpu.sync_copy(out_v, out_hbm.at[pl.ds(base, _E_PER)])


def kernel(source_nodes, target_nodes, node_features, W, b):
    src = source_nodes.astype(jnp.int32)
    tgt = target_nodes.astype(jnp.int32)
    w0 = W[:D_FEAT].reshape(1, D_FEAT)
    w1 = W[D_FEAT:].reshape(1, D_FEAT)
    b_s = b.reshape(1, 1)
    s0, s1 = _scores_call(node_features, w0, w1, b_s)
    return _edge_gather(s0, s1, src, tgt)


# speedup vs baseline: 42.9807x; 1.0644x over previous
import functools

import jax
import jax.numpy as jnp
from jax import lax
from jax.experimental import pallas as pl
from jax.experimental.pallas import tpu as pltpu
from jax.experimental.pallas import tpu_sc as plsc

N_NODES = 10000
N_EDGES = 320000
D_FEAT = 128

_NC, _NS = 2, 16
_NW = _NC * _NS
_E_PER = N_EDGES // _NW
_CHUNK = 16

def _scores_body(x_ref, w0_ref, w1_ref, b_ref, o0_ref, o1_ref):
    x = x_ref[...]
    o0_ref[...] = jnp.sum(x * w0_ref[...], axis=1) + b_ref[0, 0]
    o1_ref[...] = jnp.sum(x * w1_ref[...], axis=1)


_scores_call = pl.pallas_call(
    _scores_body,
    out_shape=[
        jax.ShapeDtypeStruct((N_NODES,), jnp.float32),
        jax.ShapeDtypeStruct((N_NODES,), jnp.float32),
    ],
)


_mesh = plsc.VectorSubcoreMesh(core_axis_name="c", subcore_axis_name="s")


@functools.partial(
    pl.kernel,
    mesh=_mesh,
    out_type=jax.ShapeDtypeStruct((N_EDGES,), jnp.float32),
    scratch_types=[
        pltpu.VMEM((N_NODES,), jnp.float32),
        pltpu.VMEM((N_NODES,), jnp.float32),
        pltpu.VMEM((_E_PER,), jnp.int32),
        pltpu.VMEM((_E_PER,), jnp.int32),
        pltpu.VMEM((_E_PER,), jnp.float32),
        pltpu.SemaphoreType.DMA,
        pltpu.SemaphoreType.DMA,
        pltpu.SemaphoreType.DMA,
        pltpu.SemaphoreType.DMA,
    ],
    compiler_params=pltpu.CompilerParams(needs_layout_passes=False),
)
def _edge_gather(
    s0_hbm, s1_hbm, src_hbm, tgt_hbm, out_hbm,
    s0_v, s1_v, src_v, tgt_v, out_v, sem0, sem1, sem2, sem3,
):
    wid = lax.axis_index("s") * _NC + lax.axis_index("c")
    base = wid * _E_PER
    cp0 = pltpu.async_copy(s0_hbm, s0_v, sem0)
    cp1 = pltpu.async_copy(s1_hbm, s1_v, sem1)
    cp2 = pltpu.async_copy(src_hbm.at[pl.ds(base, _E_PER)], src_v, sem2)
    cp3 = pltpu.async_copy(tgt_hbm.at[pl.ds(base, _E_PER)], tgt_v, sem3)
    cp0.wait()
    cp1.wait()
    cp2.wait()
    cp3.wait()

    @plsc.parallel_loop(0, _E_PER // _CHUNK, 1, unroll=8)
    def _loop(i):
        off = pl.multiple_of(i * _CHUNK, _CHUNK)
        si = src_v[pl.ds(off, _CHUNK)]
        ti = tgt_v[pl.ds(off, _CHUNK)]
        vs = plsc.load_gather(s0_v, [si])
        vt = plsc.load_gather(s1_v, [ti])
        out_v[pl.ds(off, _CHUNK)] = vs + vt
    pltpu.sync_copy(out_v, out_hbm.at[pl.ds(base, _E_PER)])


def kernel(source_nodes, target_nodes, node_features, W, b):
    src = source_nodes.astype(jnp.int32)
    tgt = target_nodes.astype(jnp.int32)
    w0 = W[:D_FEAT].reshape(1, D_FEAT)
    w1 = W[D_FEAT:].reshape(1, D_FEAT)
    b_s = b.reshape(1, 1)
    s0, s1 = _scores_call(node_features, w0, w1, b_s)
    return _edge_gather(s0, s1, src, tgt)
